# baseline (speedup 1.0000x reference)
import functools

import jax
import jax.numpy as jnp
from jax import lax
from jax.experimental import pallas as pl
from jax.experimental.pallas import tpu as pltpu
from jax.experimental.pallas import tpu_sc as plsc

NC = 2
NS = 16
NW = NC * NS
D = 32
CHUNK = 128


def _sc_gather(P, Q, ub, ib, user_id, item_id):
    B = user_id.shape[0]
    b_per_w = B // NW
    n_ch = b_per_w // CHUNK
    mesh = plsc.VectorSubcoreMesh(core_axis_name="c", subcore_axis_name="s")

    @functools.partial(
        pl.kernel,
        mesh=mesh,
        compiler_params=pltpu.CompilerParams(use_tc_tiling_on_sc=False),
        out_type=(
            jax.ShapeDtypeStruct((B, D), jnp.float32),
            jax.ShapeDtypeStruct((B, D), jnp.float32),
            jax.ShapeDtypeStruct((B,), jnp.float32),
            jax.ShapeDtypeStruct((B,), jnp.float32),
        ),
        scratch_types=[
            pltpu.VMEM((n_ch, CHUNK), jnp.int32),
            pltpu.VMEM((n_ch, CHUNK), jnp.int32),
            pltpu.VMEM((b_per_w, D), jnp.float32),
            pltpu.VMEM((b_per_w, D), jnp.float32),
            pltpu.VMEM((b_per_w,), jnp.float32),
            pltpu.VMEM((b_per_w,), jnp.float32),
            pltpu.SemaphoreType.DMA,
            pltpu.SemaphoreType.DMA,
        ],
    )
    def k(P_hbm, Q_hbm, ub_hbm, ib_hbm, uid_hbm, iid_hbm,
          pu_out, qi_out, bu_out, bi_out,
          uid_v, iid_v, pr_v, qr_v, bu_v, bi_v, sem, sem2):
        wid = lax.axis_index("s") * NC + lax.axis_index("c")
        base = wid * b_per_w
        gathers = []
        for c in range(n_ch):
            off = base + c * CHUNK
            pltpu.sync_copy(uid_hbm.at[pl.ds(off, CHUNK)], uid_v.at[c])
            pltpu.sync_copy(iid_hbm.at[pl.ds(off, CHUNK)], iid_v.at[c])
            sl = pl.ds(c * CHUNK, CHUNK)
            gathers.append(pltpu.async_copy(P_hbm.at[uid_v.at[c]], pr_v.at[sl], sem))
            gathers.append(pltpu.async_copy(Q_hbm.at[iid_v.at[c]], qr_v.at[sl], sem))
            gathers.append(pltpu.async_copy(ub_hbm.at[uid_v.at[c]], bu_v.at[sl], sem))
            gathers.append(pltpu.async_copy(ib_hbm.at[iid_v.at[c]], bi_v.at[sl], sem))
        for g in gathers:
            g.wait()
        sl_out = pl.ds(base, b_per_w)
        outs = [
            pltpu.async_copy(pr_v, pu_out.at[sl_out], sem2),
            pltpu.async_copy(qr_v, qi_out.at[sl_out], sem2),
            pltpu.async_copy(bu_v, bu_out.at[sl_out], sem2),
            pltpu.async_copy(bi_v, bi_out.at[sl_out], sem2),
        ]
        for o in outs:
            o.wait()

    return k(P, Q, ub, ib, user_id, item_id)


def _reduce_body(p_ref, q_ref, bu_ref, bi_ref, o_ref):
    o_ref[...] = jnp.sum(p_ref[...] * q_ref[...], axis=1) + bu_ref[...] + bi_ref[...]


def _tc_reduce(pu, qi, bu, bi):
    B = pu.shape[0]
    return pl.pallas_call(
        _reduce_body,
        out_shape=jax.ShapeDtypeStruct((B,), jnp.float32),
    )(pu, qi, bu, bi)


def kernel(user_id, item_id, P, Q, user_bias, item_bias):
    ub = user_bias.reshape(-1)
    ib = item_bias.reshape(-1)
    pu, qi, bu, bi = _sc_gather(P, Q, ub, ib, user_id, item_id)
    return _tc_reduce(pu, qi, bu, bi)
